# Initial kernel scaffold; baseline (speedup 1.0000x reference)
import jax
import jax.numpy as jnp
from jax.experimental import pallas as pl


def kernel(x, edge_index, edge_attr, We1, be1, We2, be2, ge, bbe, Wn1, bn1, Wn2, bn2, gn, bbn):
    raise NotImplementedError("write your pallas kernel here")


# speedup vs baseline: 3.1062x; 3.1062x over previous
import functools

import jax
import jax.numpy as jnp
from jax import lax
from jax.experimental import pallas as pl
from jax.experimental.pallas import tpu as pltpu
from jax.experimental.pallas import tpu_sc as plsc

F32 = jnp.float32

N_NODES = 10000
N_EDGES = 320000
D_FEAT = 128
D_EDGE = 16
HIDDEN = 128

NC = 2
NS = 16
NW = NC * NS

GCHUNK = 80
SCHUNK = 80

NPW = N_EDGES // NW
NPS = N_EDGES // NS
NPAD = 10240
ROWS_PT = NPAD // NS
ZROWS = 128


def _precompute_body(x_ref, wa_ref, wb_ref, b_ref, xa_ref, xb_ref):
    x = x_ref[...]
    xa_ref[...] = jnp.dot(x, wa_ref[...], preferred_element_type=F32) + b_ref[...]
    xb_ref[...] = jnp.dot(x, wb_ref[...], preferred_element_type=F32)


def _precompute(x, wa, wb, b):
    return pl.pallas_call(
        _precompute_body,
        out_shape=(
            jax.ShapeDtypeStruct((N_NODES, D_FEAT), F32),
            jax.ShapeDtypeStruct((N_NODES, D_FEAT), F32),
        ),
    )(x, wa, wb, b)


def _gather_body(xa_hbm, xb_hbm, src_hbm, dst_hbm, g_hbm,
                 bufa, bufb, idxa, idxb, sema, semb):
    c = lax.axis_index("c")
    s = lax.axis_index("s")
    wid = s * NC + c
    base0 = wid * NPW

    def body(i, carry):
        base = pl.multiple_of(base0 + i * GCHUNK, GCHUNK)
        pltpu.sync_copy(src_hbm.at[pl.ds(base, GCHUNK)], idxa)
        pltpu.sync_copy(dst_hbm.at[pl.ds(base, GCHUNK)], idxb)
        cpa = pltpu.async_copy(xa_hbm.at[idxa], bufa, sema)
        cpb = pltpu.async_copy(xb_hbm.at[idxb], bufb, semb)
        cpa.wait()
        cpb.wait()

        def row(j, carry2):
            for q in range(D_FEAT // 16):
                sl = pl.ds(q * 16, 16)
                bufa[j, sl] = bufa[j, sl] + bufb[j, sl]
            return carry2

        lax.fori_loop(0, GCHUNK, row, 0, unroll=False)
        pltpu.sync_copy(bufa, g_hbm.at[pl.ds(base, GCHUNK)])
        return carry

    lax.fori_loop(0, NPW // GCHUNK, body, 0, unroll=False)


@functools.partial(jax.jit, static_argnames=())
def _gather(xa, xb, src, dst):
    mesh = plsc.VectorSubcoreMesh(core_axis_name="c", subcore_axis_name="s")
    return pl.kernel(
        _gather_body,
        out_type=jax.ShapeDtypeStruct((N_EDGES, D_FEAT), F32),
        mesh=mesh,
        scratch_types=[
            pltpu.VMEM((GCHUNK, D_FEAT), F32),
            pltpu.VMEM((GCHUNK, D_FEAT), F32),
            pltpu.VMEM((GCHUNK,), jnp.int32),
            pltpu.VMEM((GCHUNK,), jnp.int32),
            pltpu.SemaphoreType.DMA,
            pltpu.SemaphoreType.DMA,
        ],
    )(xa, xb, src, dst)


def _edge_body(g_ref, ea_ref, wc_ref, w2_ref, b2_ref, ge_ref, bbe_ref, out_ref):
    h = g_ref[...] + jnp.dot(ea_ref[...], wc_ref[...], preferred_element_type=F32)
    h = jnp.maximum(h, 0.0)
    y = jnp.dot(h, w2_ref[...], preferred_element_type=F32) + b2_ref[...]
    mu = jnp.mean(y, axis=-1, keepdims=True)
    var = jnp.mean(jnp.square(y - mu), axis=-1, keepdims=True)
    out_ref[...] = (y - mu) * lax.rsqrt(var + 1e-5) * ge_ref[...] + bbe_ref[...]


def _edge_mlp(g, ea, wc, w2, b2, ge, bbe, blk=2000):
    nblk = N_EDGES // blk
    full = lambda i: (0, 0)
    return pl.pallas_call(
        _edge_body,
        grid=(nblk,),
        in_specs=[
            pl.BlockSpec((blk, D_FEAT), lambda i: (i, 0)),
            pl.BlockSpec((blk, D_EDGE), lambda i: (i, 0)),
            pl.BlockSpec((D_EDGE, HIDDEN), full),
            pl.BlockSpec((HIDDEN, D_FEAT), full),
            pl.BlockSpec((1, D_FEAT), full),
            pl.BlockSpec((1, D_FEAT), full),
            pl.BlockSpec((1, D_FEAT), full),
        ],
        out_specs=pl.BlockSpec((blk, D_FEAT), lambda i: (i, 0)),
        out_shape=jax.ShapeDtypeStruct((N_EDGES, D_FEAT), F32),
        compiler_params=pltpu.CompilerParams(
            dimension_semantics=("arbitrary",),
        ),
    )(g, ea, wc, w2, b2, ge, bbe)


def _scatter_body(edata_hbm, src_hbm, dst_hbm, sums_hbm, cnts_hbm,
                  buf, idxb, ones, zbuf, zcnt, acc, cacc):
    c = lax.axis_index("c")
    s = lax.axis_index("s")

    def zrow(j, carry):
        for q in range(D_FEAT // 16):
            zbuf[j, pl.ds(q * 16, 16)] = jnp.zeros((16,), F32)
        return carry

    lax.fori_loop(0, ZROWS, zrow, 0, unroll=False)

    def zcrow(j, carry):
        zcnt[j, :] = jnp.zeros((16,), F32)
        return carry

    lax.fori_loop(0, ROWS_PT, zcrow, 0, unroll=False)

    def orow(j, carry):
        ones[j, :] = jnp.ones((16,), F32)
        return carry

    lax.fori_loop(0, SCHUNK, orow, 0, unroll=False)

    for k in range(ROWS_PT // ZROWS):
        pltpu.sync_copy(zbuf, acc.at[pl.ds(s * ROWS_PT + k * ZROWS, ZROWS)])
    pltpu.sync_copy(zcnt, cacc.at[pl.ds(s * ROWS_PT, ROWS_PT)])
    plsc.subcore_barrier()

    base0 = s * NPS

    def body(i, carry):
        base = pl.multiple_of(base0 + i * SCHUNK, SCHUNK)

        @pl.when(c == 0)
        def _():
            pltpu.sync_copy(src_hbm.at[pl.ds(base, SCHUNK)], idxb)

        @pl.when(c == 1)
        def _():
            pltpu.sync_copy(dst_hbm.at[pl.ds(base, SCHUNK)], idxb)

        pltpu.sync_copy(edata_hbm.at[pl.ds(base, SCHUNK)], buf)
        pltpu.sync_copy(buf, acc.at[idxb], add=True)
        pltpu.sync_copy(ones, cacc.at[idxb], add=True)
        return carry

    lax.fori_loop(0, NPS // SCHUNK, body, 0, unroll=False)
    plsc.subcore_barrier()

    for k in range(ROWS_PT // ZROWS):
        r0 = s * ROWS_PT + k * ZROWS
        pltpu.sync_copy(acc.at[pl.ds(r0, ZROWS)], zbuf)
        pltpu.sync_copy(zbuf, sums_hbm.at[c, pl.ds(r0, ZROWS)])
    pltpu.sync_copy(cacc.at[pl.ds(s * ROWS_PT, ROWS_PT)], zcnt)
    pltpu.sync_copy(zcnt, cnts_hbm.at[c, pl.ds(s * ROWS_PT, ROWS_PT)])


def _scatter(edata, src, dst):
    mesh = plsc.VectorSubcoreMesh(core_axis_name="c", subcore_axis_name="s")
    return pl.kernel(
        _scatter_body,
        out_type=(
            jax.ShapeDtypeStruct((2, NPAD, D_FEAT), F32),
            jax.ShapeDtypeStruct((2, NPAD, 16), F32),
        ),
        mesh=mesh,
        scratch_types=[
            pltpu.VMEM((SCHUNK, D_FEAT), F32),
            pltpu.VMEM((SCHUNK,), jnp.int32),
            pltpu.VMEM((SCHUNK, 16), F32),
            pltpu.VMEM((ZROWS, D_FEAT), F32),
            pltpu.VMEM((ROWS_PT, 16), F32),
            pltpu.VMEM_SHARED((NPAD, D_FEAT), F32),
            pltpu.VMEM_SHARED((NPAD, 16), F32),
        ],
        compiler_params=pltpu.CompilerParams(use_tc_tiling_on_sc=False),
    )(edata, src, dst)


def _node_body(sums_ref, cnts_ref, x_ref, wa_ref, wb_ref, wx_ref, b1_ref,
               w2_ref, b2_ref, gn_ref, bbn_ref, out_ref):
    inv_s = 1.0 / jnp.maximum(cnts_ref[0, :, 0:1], 1.0)
    inv_d = 1.0 / jnp.maximum(cnts_ref[1, :, 0:1], 1.0)
    agg_s = sums_ref[0] * inv_s
    agg_d = sums_ref[1] * inv_d
    h = (jnp.dot(agg_s, wa_ref[...], preferred_element_type=F32)
         + jnp.dot(agg_d, wb_ref[...], preferred_element_type=F32)
         + jnp.dot(x_ref[...], wx_ref[...], preferred_element_type=F32)
         + b1_ref[...])
    h = jnp.maximum(h, 0.0)
    y = jnp.dot(h, w2_ref[...], preferred_element_type=F32) + b2_ref[...]
    mu = jnp.mean(y, axis=-1, keepdims=True)
    var = jnp.mean(jnp.square(y - mu), axis=-1, keepdims=True)
    out_ref[...] = (y - mu) * lax.rsqrt(var + 1e-5) * gn_ref[...] + bbn_ref[...]


def _node_mlp(sums, cnts, x, wa, wb, wx, b1, w2, b2, gn, bbn):
    return pl.pallas_call(
        _node_body,
        out_shape=jax.ShapeDtypeStruct((N_NODES, D_FEAT), F32),
    )(sums, cnts, x, wa, wb, wx, b1, w2, b2, gn, bbn)


def kernel(x, edge_index, edge_attr, We1, be1, We2, be2, ge, bbe,
           Wn1, bn1, Wn2, bn2, gn, bbn):
    src = edge_index[0]
    dst = edge_index[1]

    xa, xb = _precompute(x, We1[:D_FEAT], We1[D_FEAT:2 * D_FEAT],
                         be1.reshape(1, -1))
    g = _gather(xa, xb, src, dst)
    edata = _edge_mlp(g, edge_attr, We1[2 * D_FEAT:], We2,
                      be2.reshape(1, -1), ge.reshape(1, -1),
                      bbe.reshape(1, -1))
    sums, cnts = _scatter(edata, src, dst)
    sums = sums[:, :N_NODES]
    cnts = cnts[:, :N_NODES]
    ndata = _node_mlp(sums, cnts, x,
                      Wn1[:D_FEAT], Wn1[D_FEAT:2 * D_FEAT], Wn1[2 * D_FEAT:],
                      bn1.reshape(1, -1), Wn2, bn2.reshape(1, -1),
                      gn.reshape(1, -1), bbn.reshape(1, -1))
    return (ndata, edata)
